# Initial kernel scaffold; baseline (speedup 1.0000x reference)
import jax
import jax.numpy as jnp
from jax.experimental import pallas as pl


def kernel(qkv, seq_offsets, timestamps, tw, pw):
    raise NotImplementedError("write your pallas kernel here")


# speedup vs baseline: 17.7639x; 17.7639x over previous
import jax
import jax.numpy as jnp
from jax.experimental import pallas as pl
from jax.experimental.pallas import tpu as pltpu

N_MAX = 256
N_HEADS = 4
D_HEAD = 128
ALPHA = 0.08838834764831843
ROW_F = N_HEADS * 3 * D_HEAD
OUT_F = N_HEADS * D_HEAD


def _block_kernel(xc_ref, xp_ref, rec_ref, rep_ref, o_ref, *, nb, lp_rows):
    b = pl.program_id(0)
    xc = xc_ref[...]
    xp = xp_ref[...]

    ke = jnp.concatenate([rep_ref[0], rec_ref[0]], axis=1)

    gi = jax.lax.broadcasted_iota(jnp.int32, (N_MAX, 2 * N_MAX), 0)
    ci = jax.lax.broadcasted_iota(jnp.int32, (N_MAX, 2 * N_MAX), 1)
    base = b * N_MAX
    mask = (ci <= gi + N_MAX) & (ci + (base - N_MAX) >= 0) & (gi + base < ke)

    vrow = jax.lax.broadcasted_iota(jnp.int32, (2 * N_MAX, D_HEAD), 0)
    vok = vrow + (base - N_MAX) < lp_rows

    for h in range(N_HEADS):
        o = h * 3 * D_HEAD
        q = xc[:, o:o + D_HEAD]
        k = jnp.concatenate(
            [xp[:, o + D_HEAD:o + 2 * D_HEAD], xc[:, o + D_HEAD:o + 2 * D_HEAD]],
            axis=0)
        v = jnp.concatenate(
            [xp[:, o + 2 * D_HEAD:o + 3 * D_HEAD], xc[:, o + 2 * D_HEAD:o + 3 * D_HEAD]],
            axis=0)
        v = jnp.where(vok, v, jnp.bfloat16(0))
        s = jax.lax.dot_general(q, k, (((1,), (1,)), ((), ())),
                                preferred_element_type=jnp.float32)
        s = s * ALPHA
        a = s * jax.nn.sigmoid(s) * (1.0 / N_MAX)
        a = jnp.where(mask, a, 0.0).astype(jnp.bfloat16)
        out = jax.lax.dot_general(a, v, (((1,), (0,)), ((), ())),
                                  preferred_element_type=jnp.float32)
        o_ref[:, h * D_HEAD:(h + 1) * D_HEAD] = out.astype(jnp.bfloat16)


@jax.jit
def kernel(qkv, seq_offsets, timestamps, tw, pw):
    L = qkv.shape[0]
    nb = (L + N_MAX - 1) // N_MAX
    x = qkv.reshape(L, ROW_F)

    offs = seq_offsets.astype(jnp.int32)
    lengths = offs[1:] - offs[:-1]
    row_end = jnp.repeat(offs[1:], lengths, total_repeat_length=L)
    row_end = jnp.pad(row_end, (0, nb * N_MAX - L))
    re3 = row_end.reshape(nb, 1, N_MAX)

    import functools
    out = pl.pallas_call(
        functools.partial(_block_kernel, nb=nb, lp_rows=L),
        grid=(nb,),
        in_specs=[
            pl.BlockSpec((N_MAX, ROW_F), lambda b: (b, 0)),
            pl.BlockSpec((N_MAX, ROW_F), lambda b: (jnp.maximum(b - 1, 0), 0)),
            pl.BlockSpec((1, 1, N_MAX), lambda b: (b, 0, 0)),
            pl.BlockSpec((1, 1, N_MAX), lambda b: (jnp.maximum(b - 1, 0), 0, 0)),
        ],
        out_specs=pl.BlockSpec((N_MAX, OUT_F), lambda b: (b, 0)),
        out_shape=jax.ShapeDtypeStruct((L, OUT_F), jnp.bfloat16),
        compiler_params=pltpu.CompilerParams(
            dimension_semantics=("parallel",),
        ),
    )(x, x, re3, re3)
    return out.reshape(L, N_HEADS, D_HEAD)
